# Initial kernel scaffold; baseline (speedup 1.0000x reference)
import jax
import jax.numpy as jnp
from jax.experimental import pallas as pl


def kernel(x, edge_index, batch, W_gnn, b_gnn, W1, b1, W2, b2):
    raise NotImplementedError("write your pallas kernel here")


# speedup vs baseline: 3.1303x; 3.1303x over previous
import functools

import jax
import jax.numpy as jnp
from jax import lax
from jax.experimental import pallas as pl
from jax.experimental.pallas import tpu as pltpu
from jax.experimental.pallas import tpu_sc as plsc

N = 10000
E = 320000
D = 128
H = 128
C = 10
G = 64

NCORE = 2
NSUB = 16
NT = NCORE * NSUB
K = 128
EPT = 10240
NCHUNK = EPT // K
IB = 8
NBLK = NCHUNK // IB
E_PAD = EPT * NT
N2 = 10240


def _sc_body(x_hbm, srcg, dstg, zacc, ones_hbm,
             acc_out, deg_out,
             acc_sh, src_v, dst_v, rows_v, ones_v, sem):
    cid = lax.axis_index("c")
    sid = lax.axis_index("s")
    wid = cid * NSUB + sid

    @pl.when(sid == 0)
    def _init():
        pltpu.sync_copy(zacc, acc_sh)

    pltpu.sync_copy(ones_hbm, ones_v)
    plsc.subcore_barrier()

    def step(j, c):
        pltpu.sync_copy(srcg.at[wid, j], src_v)
        pltpu.sync_copy(dstg.at[wid, j], dst_v)
        pltpu.async_copy(x_hbm.at[src_v], rows_v, sem).wait()
        pltpu.sync_copy(rows_v, acc_sh.at[dst_v], add=True)
        return c

    lax.fori_loop(0, NCHUNK, step, 0)
    plsc.subcore_barrier()

    @pl.when(sid == 0)
    def _flush_acc():
        pltpu.sync_copy(acc_sh, acc_out.at[cid])
        pltpu.sync_copy(zacc, acc_sh)

    plsc.subcore_barrier()

    def dstep(j, c):
        pltpu.sync_copy(dstg.at[wid, j], dst_v)
        pltpu.sync_copy(ones_v, acc_sh.at[dst_v], add=True)
        return c

    lax.fori_loop(0, NCHUNK, dstep, 0)
    plsc.subcore_barrier()

    @pl.when(sid == 0)
    def _flush_deg():
        pltpu.sync_copy(acc_sh, deg_out.at[cid])


def _tc_body(acc_ref, deg_ref, batch_ref, wg_ref, bg_ref, w1_ref, b1_ref,
             w2_ref, b2_ref, out_ref):
    prec = lax.Precision.HIGHEST
    acc = acc_ref[0] + acc_ref[1]
    deg = deg_ref[0, :, 0:1] + deg_ref[1, :, 0:1]
    degc = jnp.maximum(deg, 1.0)
    agg = acc / degc
    h = jnp.maximum(
        jnp.dot(agg, wg_ref[...], precision=prec,
                preferred_element_type=jnp.float32) + bg_ref[...], 0.0)

    onehot = (batch_ref[...] ==
              lax.broadcasted_iota(jnp.int32, (G, N2), 0)).astype(jnp.float32)
    pooled = jnp.dot(onehot, h, precision=prec,
                     preferred_element_type=jnp.float32)
    cnt = jnp.dot(onehot, jnp.full((N2, 1), 1.0, jnp.float32),
                  precision=prec, preferred_element_type=jnp.float32)
    gmean = pooled / jnp.maximum(cnt, 1.0)

    h1 = jnp.maximum(
        jnp.dot(gmean, w1_ref[...], precision=prec,
                preferred_element_type=jnp.float32) + b1_ref[...], 0.0)
    out_ref[...] = jnp.dot(h1, w2_ref[...], precision=prec,
                           preferred_element_type=jnp.float32) + b2_ref[...]


@functools.lru_cache(maxsize=1)
def _sc_aggregate():
    return pl.kernel(
        _sc_body,
        out_type=(jax.ShapeDtypeStruct((NCORE, N2, D), jnp.float32),
                  jax.ShapeDtypeStruct((NCORE, N2, D), jnp.float32)),
        mesh=plsc.VectorSubcoreMesh(core_axis_name="c",
                                    subcore_axis_name="s"),
        scratch_types=[
            pltpu.VMEM_SHARED((N2, D), jnp.float32),
            pltpu.VMEM((K,), jnp.int32),
            pltpu.VMEM((K,), jnp.int32),
            pltpu.VMEM((K, D), jnp.float32),
            pltpu.VMEM((K, D), jnp.float32),
            pltpu.SemaphoreType.DMA,
        ],
    )


_tc_head = pl.pallas_call(
    _tc_body,
    out_shape=jax.ShapeDtypeStruct((G, C), jnp.float32),
)


def kernel(x, edge_index, batch, W_gnn, b_gnn, W1, b1, W2, b2):
    src = edge_index[0]
    dst = edge_index[1]
    pad = E_PAD - E
    srcg = jnp.concatenate(
        [src, jnp.zeros((pad,), jnp.int32)]).reshape(NT, NCHUNK, K)
    dstg = jnp.concatenate(
        [dst, jnp.full((pad,), N, jnp.int32)]).reshape(NT, NCHUNK, K)
    zacc = jnp.zeros((N2, D), jnp.float32)
    ones = jnp.ones((K, D), jnp.float32)

    acc_part, deg_part = _sc_aggregate()(x, srcg, dstg, zacc, ones)

    batch2 = jnp.concatenate(
        [batch, jnp.full((N2 - N,), G, jnp.int32)]).reshape(1, N2)
    return _tc_head(acc_part, deg_part, batch2,
                    W_gnn, b_gnn.reshape(1, D), W1, b1.reshape(1, H),
                    W2, b2.reshape(1, C))


# speedup vs baseline: 3.9123x; 1.2498x over previous
import functools

import jax
import jax.numpy as jnp
from jax import lax
from jax.experimental import pallas as pl
from jax.experimental.pallas import tpu as pltpu
from jax.experimental.pallas import tpu_sc as plsc

N = 10000
E = 320000
D = 128
H = 128
C = 10
G = 64

NCORE = 2
NSUB = 16
NT = NCORE * NSUB
K = 128
EPT = 10240
NCHUNK = EPT // K
IB = 8
NBLK = NCHUNK // IB
E_PAD = EPT * NT
N2 = 10240


def _sc_body(x_hbm, srcg, dstg, zacc, ones_hbm,
             acc_out, deg_out,
             acc_sh, src0, src1, dst0, dst1, rows0, rows1,
             sem_i0, sem_i1, sem_g0, sem_g1):
    cid = lax.axis_index("c")
    sid = lax.axis_index("s")
    wid = cid * NSUB + sid
    srcs = (src0, src1)
    dsts = (dst0, dst1)
    rows = (rows0, rows1)
    sem_i = (sem_i0, sem_i1)
    sem_g = (sem_g0, sem_g1)

    @pl.when(sid == 0)
    def _init():
        pltpu.sync_copy(zacc, acc_sh)

    plsc.subcore_barrier()

    def fire_idx(j, p):
        pltpu.async_copy(srcg.at[wid, j], srcs[p], sem_i[p])
        pltpu.async_copy(dstg.at[wid, j], dsts[p], sem_i[p])

    def wait_idx(p):
        pltpu.make_async_copy(srcg.at[wid, 0], srcs[p], sem_i[p]).wait()
        pltpu.make_async_copy(dstg.at[wid, 0], dsts[p], sem_i[p]).wait()

    fire_idx(0, 0)
    wait_idx(0)
    pltpu.async_copy(x_hbm.at[src0], rows0, sem_g0)
    fire_idx(1, 1)

    def half(j, p):
        q = 1 - p

        @pl.when(j + 1 < NCHUNK)
        def _():
            wait_idx(q)
            pltpu.async_copy(x_hbm.at[srcs[q]], rows[q], sem_g[q])

        pltpu.make_async_copy(x_hbm.at[srcs[p]], rows[p], sem_g[p]).wait()
        pltpu.sync_copy(rows[p], acc_sh.at[dsts[p]], add=True)

        @pl.when(j + 2 < NCHUNK)
        def _():
            fire_idx(j + 2, p)

    def step(t, c):
        half(2 * t, 0)
        half(2 * t + 1, 1)
        return c

    lax.fori_loop(0, NCHUNK // 2, step, 0)
    plsc.subcore_barrier()

    @pl.when(sid == 0)
    def _flush_acc():
        pltpu.sync_copy(acc_sh, acc_out.at[cid])
        pltpu.sync_copy(zacc, acc_sh)

    pltpu.sync_copy(ones_hbm, rows0)
    plsc.subcore_barrier()

    pltpu.async_copy(dstg.at[wid, 0], dst0, sem_i0)

    def dhalf(j, p):
        q = 1 - p

        @pl.when(j + 1 < NCHUNK)
        def _():
            pltpu.async_copy(dstg.at[wid, j + 1], dsts[q], sem_i[q])

        pltpu.make_async_copy(dstg.at[wid, 0], dsts[p], sem_i[p]).wait()
        pltpu.sync_copy(rows0, acc_sh.at[dsts[p]], add=True)

    def dstep(t, c):
        dhalf(2 * t, 0)
        dhalf(2 * t + 1, 1)
        return c

    lax.fori_loop(0, NCHUNK // 2, dstep, 0)
    plsc.subcore_barrier()

    @pl.when(sid == 0)
    def _flush_deg():
        pltpu.sync_copy(acc_sh, deg_out.at[cid])


def _tc_body(acc_ref, deg_ref, batch_ref, wg_ref, bg_ref, w1_ref, b1_ref,
             w2_ref, b2_ref, out_ref):
    prec = lax.Precision.HIGHEST
    acc = acc_ref[0] + acc_ref[1]
    deg = deg_ref[0, :, 0:1] + deg_ref[1, :, 0:1]
    degc = jnp.maximum(deg, 1.0)
    agg = acc / degc
    h = jnp.maximum(
        jnp.dot(agg, wg_ref[...], precision=prec,
                preferred_element_type=jnp.float32) + bg_ref[...], 0.0)

    onehot = (batch_ref[...] ==
              lax.broadcasted_iota(jnp.int32, (G, N2), 0)).astype(jnp.float32)
    pooled = jnp.dot(onehot, h, precision=prec,
                     preferred_element_type=jnp.float32)
    cnt = jnp.dot(onehot, jnp.full((N2, 1), 1.0, jnp.float32),
                  precision=prec, preferred_element_type=jnp.float32)
    gmean = pooled / jnp.maximum(cnt, 1.0)

    h1 = jnp.maximum(
        jnp.dot(gmean, w1_ref[...], precision=prec,
                preferred_element_type=jnp.float32) + b1_ref[...], 0.0)
    out_ref[...] = jnp.dot(h1, w2_ref[...], precision=prec,
                           preferred_element_type=jnp.float32) + b2_ref[...]


@functools.lru_cache(maxsize=1)
def _sc_aggregate():
    return pl.kernel(
        _sc_body,
        out_type=(jax.ShapeDtypeStruct((NCORE, N2, D), jnp.float32),
                  jax.ShapeDtypeStruct((NCORE, N2, D), jnp.float32)),
        mesh=plsc.VectorSubcoreMesh(core_axis_name="c",
                                    subcore_axis_name="s"),
        scratch_types=[
            pltpu.VMEM_SHARED((N2, D), jnp.float32),
            pltpu.VMEM((K,), jnp.int32),
            pltpu.VMEM((K,), jnp.int32),
            pltpu.VMEM((K,), jnp.int32),
            pltpu.VMEM((K,), jnp.int32),
            pltpu.VMEM((K, D), jnp.float32),
            pltpu.VMEM((K, D), jnp.float32),
            pltpu.SemaphoreType.DMA,
            pltpu.SemaphoreType.DMA,
            pltpu.SemaphoreType.DMA,
            pltpu.SemaphoreType.DMA,
        ],
    )


_tc_head = pl.pallas_call(
    _tc_body,
    out_shape=jax.ShapeDtypeStruct((G, C), jnp.float32),
)


def kernel(x, edge_index, batch, W_gnn, b_gnn, W1, b1, W2, b2):
    src = edge_index[0]
    dst = edge_index[1]
    pad = E_PAD - E
    srcg = jnp.concatenate(
        [src, jnp.zeros((pad,), jnp.int32)]).reshape(NT, NCHUNK, K)
    dstg = jnp.concatenate(
        [dst, jnp.full((pad,), N, jnp.int32)]).reshape(NT, NCHUNK, K)
    zacc = jnp.zeros((N2, D), jnp.float32)
    ones = jnp.ones((K, D), jnp.float32)

    acc_part, deg_part = _sc_aggregate()(x, srcg, dstg, zacc, ones)

    batch2 = jnp.concatenate(
        [batch, jnp.full((N2 - N,), G, jnp.int32)]).reshape(1, N2)
    return _tc_head(acc_part, deg_part, batch2,
                    W_gnn, b_gnn.reshape(1, D), W1, b1.reshape(1, H),
                    W2, b2.reshape(1, C))


# speedup vs baseline: 3.9144x; 1.0006x over previous
import functools

import jax
import jax.numpy as jnp
from jax import lax
from jax.experimental import pallas as pl
from jax.experimental.pallas import tpu as pltpu
from jax.experimental.pallas import tpu_sc as plsc

N = 10000
E = 320000
D = 128
H = 128
C = 10
G = 64

NCORE = 2
NSUB = 16
NT = NCORE * NSUB
K = 128
EPT = 10240
NCHUNK = EPT // K
IB = 8
NBLK = NCHUNK // IB
E_PAD = EPT * NT
N2 = 10240


def _sc_body(x_hbm, srcg, dstg, zacc, ones_hbm,
             acc_out, deg_out,
             acc_sh, src0, src1, dst0, dst1, rows0, rows1,
             sem_i0, sem_i1, sem_g0, sem_g1):
    cid = lax.axis_index("c")
    sid = lax.axis_index("s")
    wid = cid * NSUB + sid
    srcs = (src0, src1)
    dsts = (dst0, dst1)
    rows = (rows0, rows1)
    sem_i = (sem_i0, sem_i1)
    sem_g = (sem_g0, sem_g1)

    @pl.when(sid == 0)
    def _init():
        pltpu.sync_copy(zacc, acc_sh)

    plsc.subcore_barrier()

    def fire_idx(j, p):
        pltpu.async_copy(srcg.at[wid, j], srcs[p], sem_i[p])
        pltpu.async_copy(dstg.at[wid, j], dsts[p], sem_i[p])

    def wait_idx(p):
        pltpu.make_async_copy(srcg.at[wid, 0], srcs[p], sem_i[p]).wait()
        pltpu.make_async_copy(dstg.at[wid, 0], dsts[p], sem_i[p]).wait()

    fire_idx(0, 0)
    wait_idx(0)
    pltpu.async_copy(x_hbm.at[src0], rows0, sem_g0)
    fire_idx(1, 1)

    def half(j, p):
        q = 1 - p

        @pl.when(j + 1 < NCHUNK)
        def _():
            wait_idx(q)
            pltpu.async_copy(x_hbm.at[srcs[q]], rows[q], sem_g[q])

        pltpu.make_async_copy(x_hbm.at[srcs[p]], rows[p], sem_g[p]).wait()
        pltpu.sync_copy(rows[p], acc_sh.at[dsts[p]], add=True)

        @pl.when(j + 2 < NCHUNK)
        def _():
            fire_idx(j + 2, p)

    def step(t, c):
        half(2 * t, 0)
        half(2 * t + 1, 1)
        return c

    with jax.named_scope("sc_phase1_features"):
        lax.fori_loop(0, NCHUNK // 2, step, 0)
    plsc.subcore_barrier()

    @pl.when(sid == 0)
    def _flush_acc():
        pltpu.sync_copy(acc_sh, acc_out.at[cid])
        pltpu.sync_copy(zacc, acc_sh)

    pltpu.sync_copy(ones_hbm, rows0)
    plsc.subcore_barrier()

    pltpu.async_copy(dstg.at[wid, 0], dst0, sem_i0)

    def dhalf(j, p):
        q = 1 - p

        @pl.when(j + 1 < NCHUNK)
        def _():
            pltpu.async_copy(dstg.at[wid, j + 1], dsts[q], sem_i[q])

        pltpu.make_async_copy(dstg.at[wid, 0], dsts[p], sem_i[p]).wait()
        pltpu.sync_copy(rows0, acc_sh.at[dsts[p]], add=True)

    def dstep(t, c):
        dhalf(2 * t, 0)
        dhalf(2 * t + 1, 1)
        return c

    with jax.named_scope("sc_phase2_degrees"):
        lax.fori_loop(0, NCHUNK // 2, dstep, 0)
    plsc.subcore_barrier()

    @pl.when(sid == 0)
    def _flush_deg():
        pltpu.sync_copy(acc_sh, deg_out.at[cid])


def _tc_body(acc_ref, deg_ref, batch_ref, wg_ref, bg_ref, w1_ref, b1_ref,
             w2_ref, b2_ref, out_ref):
    prec = lax.Precision.HIGHEST
    acc = acc_ref[0] + acc_ref[1]
    deg = deg_ref[0, :, 0:1] + deg_ref[1, :, 0:1]
    degc = jnp.maximum(deg, 1.0)
    agg = acc / degc
    h = jnp.maximum(
        jnp.dot(agg, wg_ref[...], precision=prec,
                preferred_element_type=jnp.float32) + bg_ref[...], 0.0)

    onehot = (batch_ref[...] ==
              lax.broadcasted_iota(jnp.int32, (G, N2), 0)).astype(jnp.float32)
    pooled = jnp.dot(onehot, h, precision=prec,
                     preferred_element_type=jnp.float32)
    cnt = jnp.dot(onehot, jnp.full((N2, 1), 1.0, jnp.float32),
                  precision=prec, preferred_element_type=jnp.float32)
    gmean = pooled / jnp.maximum(cnt, 1.0)

    h1 = jnp.maximum(
        jnp.dot(gmean, w1_ref[...], precision=prec,
                preferred_element_type=jnp.float32) + b1_ref[...], 0.0)
    out_ref[...] = jnp.dot(h1, w2_ref[...], precision=prec,
                           preferred_element_type=jnp.float32) + b2_ref[...]


@functools.lru_cache(maxsize=1)
def _sc_aggregate():
    return pl.kernel(
        _sc_body,
        out_type=(jax.ShapeDtypeStruct((NCORE, N2, D), jnp.float32),
                  jax.ShapeDtypeStruct((NCORE, N2, D), jnp.float32)),
        mesh=plsc.VectorSubcoreMesh(core_axis_name="c",
                                    subcore_axis_name="s"),
        scratch_types=[
            pltpu.VMEM_SHARED((N2, D), jnp.float32),
            pltpu.VMEM((K,), jnp.int32),
            pltpu.VMEM((K,), jnp.int32),
            pltpu.VMEM((K,), jnp.int32),
            pltpu.VMEM((K,), jnp.int32),
            pltpu.VMEM((K, D), jnp.float32),
            pltpu.VMEM((K, D), jnp.float32),
            pltpu.SemaphoreType.DMA,
            pltpu.SemaphoreType.DMA,
            pltpu.SemaphoreType.DMA,
            pltpu.SemaphoreType.DMA,
        ],
    )


_tc_head = pl.pallas_call(
    _tc_body,
    out_shape=jax.ShapeDtypeStruct((G, C), jnp.float32),
)


def kernel(x, edge_index, batch, W_gnn, b_gnn, W1, b1, W2, b2):
    src = edge_index[0]
    dst = edge_index[1]
    pad = E_PAD - E
    srcg = jnp.concatenate(
        [src, jnp.zeros((pad,), jnp.int32)]).reshape(NT, NCHUNK, K)
    dstg = jnp.concatenate(
        [dst, jnp.full((pad,), N, jnp.int32)]).reshape(NT, NCHUNK, K)
    zacc = jnp.zeros((N2, D), jnp.float32)
    ones = jnp.ones((K, D), jnp.float32)

    acc_part, deg_part = _sc_aggregate()(x, srcg, dstg, zacc, ones)

    batch2 = jnp.concatenate(
        [batch, jnp.full((N2 - N,), G, jnp.int32)]).reshape(1, N2)
    return _tc_head(acc_part, deg_part, batch2,
                    W_gnn, b_gnn.reshape(1, D), W1, b1.reshape(1, H),
                    W2, b2.reshape(1, C))


# speedup vs baseline: 9.2172x; 2.3547x over previous
import functools

import jax
import jax.numpy as jnp
from jax import lax
from jax.experimental import pallas as pl
from jax.experimental.pallas import tpu as pltpu
from jax.experimental.pallas import tpu_sc as plsc

N = 10000
E = 320000
D = 128
H = 128
C = 10
G = 64

NCORE = 2
NSUB = 16
NT = NCORE * NSUB
K = 128
EPT = 10240
NCHUNK = EPT // K
IB = 8
NBLK = NCHUNK // IB
E_PAD = EPT * NT
N2 = 10240


def _sc_body(x_hbm, srcg, dstg, zacc, ones_hbm,
             acc_out, deg_out,
             acc_sh, src0, src1, dst0, dst1, rows0, rows1,
             sem_i0, sem_i1, sem_g0, sem_g1):
    cid = lax.axis_index("c")
    sid = lax.axis_index("s")
    wid = cid * NSUB + sid
    srcs = (src0, src1)
    dsts = (dst0, dst1)
    rows = (rows0, rows1)
    sem_i = (sem_i0, sem_i1)
    sem_g = (sem_g0, sem_g1)

    @pl.when(sid == 0)
    def _init():
        pltpu.sync_copy(zacc, acc_sh)

    plsc.subcore_barrier()

    def fire_idx(j, p):
        pltpu.async_copy(srcg.at[wid, j], srcs[p], sem_i[p])
        pltpu.async_copy(dstg.at[wid, j], dsts[p], sem_i[p])

    def wait_idx(p):
        pltpu.make_async_copy(srcg.at[wid, 0], srcs[p], sem_i[p]).wait()
        pltpu.make_async_copy(dstg.at[wid, 0], dsts[p], sem_i[p]).wait()

    fire_idx(0, 0)
    wait_idx(0)
    pltpu.async_copy(x_hbm.at[src0], rows0, sem_g0)
    fire_idx(1, 1)

    def half(j, p):
        q = 1 - p

        @pl.when(j + 1 < NCHUNK)
        def _():
            wait_idx(q)
            pltpu.async_copy(x_hbm.at[srcs[q]], rows[q], sem_g[q])

        pltpu.make_async_copy(x_hbm.at[srcs[p]], rows[p], sem_g[p]).wait()
        pltpu.sync_copy(rows[p], acc_sh.at[dsts[p]], add=True)

        @pl.when(j + 2 < NCHUNK)
        def _():
            fire_idx(j + 2, p)

    def step(t, c):
        half(2 * t, 0)
        half(2 * t + 1, 1)
        return c

    with jax.named_scope("sc_phase1_features"):
        lax.fori_loop(0, NCHUNK // 2, step, 0)
    plsc.subcore_barrier()

    @pl.when(sid == 0)
    def _flush_acc():
        pltpu.sync_copy(acc_sh, acc_out.at[cid])
        pltpu.sync_copy(zacc, acc_sh)

    pltpu.sync_copy(ones_hbm, rows0)
    plsc.subcore_barrier()

    pltpu.async_copy(dstg.at[wid, 0], dst0, sem_i0)

    def dhalf(j, p):
        q = 1 - p

        @pl.when(j + 1 < NCHUNK)
        def _():
            pltpu.async_copy(dstg.at[wid, j + 1], dsts[q], sem_i[q])

        pltpu.make_async_copy(dstg.at[wid, 0], dsts[p], sem_i[p]).wait()
        pltpu.sync_copy(rows0, acc_sh.at[dsts[p]], add=True)

    def dstep(t, c):
        dhalf(2 * t, 0)
        dhalf(2 * t + 1, 1)
        return c

    with jax.named_scope("sc_phase2_degrees"):
        lax.fori_loop(0, NCHUNK // 2, dstep, 0)
    plsc.subcore_barrier()

    @pl.when(sid == 0)
    def _flush_deg():
        pltpu.sync_copy(acc_sh, deg_out.at[cid])


def _tc_body(acc_ref, deg_ref, batch_ref, wg_ref, bg_ref, w1_ref, b1_ref,
             w2_ref, b2_ref, out_ref):
    prec = lax.Precision.HIGHEST
    acc = acc_ref[0] + acc_ref[1]
    deg = deg_ref[0, :, 0:1] + deg_ref[1, :, 0:1]
    degc = jnp.maximum(deg, 1.0)
    agg = acc / degc
    h = jnp.maximum(
        jnp.dot(agg, wg_ref[...], precision=prec,
                preferred_element_type=jnp.float32) + bg_ref[...], 0.0)

    onehot = (batch_ref[...] ==
              lax.broadcasted_iota(jnp.int32, (G, N2), 0)).astype(jnp.float32)
    pooled = jnp.dot(onehot, h, precision=prec,
                     preferred_element_type=jnp.float32)
    cnt = jnp.dot(onehot, jnp.full((N2, 1), 1.0, jnp.float32),
                  precision=prec, preferred_element_type=jnp.float32)
    gmean = pooled / jnp.maximum(cnt, 1.0)

    h1 = jnp.maximum(
        jnp.dot(gmean, w1_ref[...], precision=prec,
                preferred_element_type=jnp.float32) + b1_ref[...], 0.0)
    out_ref[...] = jnp.dot(h1, w2_ref[...], precision=prec,
                           preferred_element_type=jnp.float32) + b2_ref[...]


@functools.lru_cache(maxsize=1)
def _sc_aggregate():
    return pl.kernel(
        _sc_body,
        out_type=(jax.ShapeDtypeStruct((NCORE, N2, D), jnp.float32),
                  jax.ShapeDtypeStruct((NCORE, N2, D), jnp.float32)),
        mesh=plsc.VectorSubcoreMesh(core_axis_name="c",
                                    subcore_axis_name="s"),
        scratch_types=[
            pltpu.VMEM_SHARED((N2, D), jnp.float32),
            pltpu.VMEM((K,), jnp.int32),
            pltpu.VMEM((K,), jnp.int32),
            pltpu.VMEM((K,), jnp.int32),
            pltpu.VMEM((K,), jnp.int32),
            pltpu.VMEM((K, D), jnp.float32),
            pltpu.VMEM((K, D), jnp.float32),
            pltpu.SemaphoreType.DMA,
            pltpu.SemaphoreType.DMA,
            pltpu.SemaphoreType.DMA,
            pltpu.SemaphoreType.DMA,
        ],
    )


_tc_head = pl.pallas_call(
    _tc_body,
    out_shape=jax.ShapeDtypeStruct((G, C), jnp.float32),
)


def kernel(x, edge_index, batch, W_gnn, b_gnn, W1, b1, W2, b2):
    src = edge_index[0]
    dst = edge_index[1]
    ppt = EPT - E // NT
    rpt = (N2 - N) // NSUB
    pad_src = jnp.broadcast_to(
        (jnp.arange(ppt, dtype=jnp.int32) * 37) % N, (NT, ppt))
    tile_in_core = (jnp.arange(NT, dtype=jnp.int32) % NSUB)
    pad_dst = (N + tile_in_core[:, None] * rpt
               + jnp.arange(ppt, dtype=jnp.int32)[None, :] % rpt)
    srcg = jnp.concatenate(
        [src.reshape(NT, E // NT), pad_src], axis=1).reshape(NT, NCHUNK, K)
    dstg = jnp.concatenate(
        [dst.reshape(NT, E // NT), pad_dst.astype(jnp.int32)],
        axis=1).reshape(NT, NCHUNK, K)
    zacc = jnp.zeros((N2, D), jnp.float32)
    ones = jnp.ones((K, D), jnp.float32)

    acc_part, deg_part = _sc_aggregate()(x, srcg, dstg, zacc, ones)

    batch2 = jnp.concatenate(
        [batch, jnp.full((N2 - N,), G, jnp.int32)]).reshape(1, N2)
    return _tc_head(acc_part, deg_part, batch2,
                    W_gnn, b_gnn.reshape(1, D), W1, b1.reshape(1, H),
                    W2, b2.reshape(1, C))


# speedup vs baseline: 9.4104x; 1.0210x over previous
import functools

import jax
import jax.numpy as jnp
from jax import lax
from jax.experimental import pallas as pl
from jax.experimental.pallas import tpu as pltpu
from jax.experimental.pallas import tpu_sc as plsc

N = 10000
E = 320000
D = 128
H = 128
C = 10
G = 64

NCORE = 2
NSUB = 16
NT = NCORE * NSUB
K = 128
EPT = 10240
NCHUNK = EPT // K
IB = 8
NBLK = NCHUNK // IB
E_PAD = EPT * NT
N2 = 10240


RPT = E // NT
NFULL = RPT // K
RTAIL = RPT - NFULL * K
PPT = EPT - RPT
RSLC = N2 // NSUB


def _sc_body(x_hbm, src_h, dst_h, psrc_h, pdst_h, zacc, ones_hbm,
             acc_out, deg_out,
             acc_sh, src0, src1, dst0, dst1, rows0, rows1,
             sem_i0, sem_i1, sem_g0, sem_g1):
    cid = lax.axis_index("c")
    sid = lax.axis_index("s")
    wid = cid * NSUB + sid
    srcs = (src0, src1)
    dsts = (dst0, dst1)
    rows = (rows0, rows1)
    sem_i = (sem_i0, sem_i1)
    sem_g = (sem_g0, sem_g1)
    rslice = pl.ds(sid * RSLC, RSLC)

    pltpu.sync_copy(zacc.at[rslice], acc_sh.at[rslice])
    plsc.subcore_barrier()

    def fire_one(j, p, real_h, pad_h, bufs):
        @pl.when(j < NFULL)
        def _():
            pltpu.async_copy(real_h.at[pl.ds(wid * RPT + j * K, K)],
                             bufs[p], sem_i[p])

        @pl.when(j == NFULL)
        def _():
            pltpu.async_copy(real_h.at[pl.ds(wid * RPT + NFULL * K, RTAIL)],
                             bufs[p].at[pl.ds(0, RTAIL)], sem_i[p])
            pltpu.async_copy(pad_h.at[pl.ds(wid * PPT, K - RTAIL)],
                             bufs[p].at[pl.ds(RTAIL, K - RTAIL)], sem_i[p])

        @pl.when(j == NFULL + 1)
        def _():
            pltpu.async_copy(pad_h.at[pl.ds(wid * PPT + K - RTAIL, K)],
                             bufs[p], sem_i[p])

    def fire_idx(j, p):
        fire_one(j, p, src_h, psrc_h, srcs)
        fire_one(j, p, dst_h, pdst_h, dsts)

    def wait_idx(p):
        pltpu.make_async_copy(src_h.at[pl.ds(0, K)], srcs[p], sem_i[p]).wait()
        pltpu.make_async_copy(dst_h.at[pl.ds(0, K)], dsts[p], sem_i[p]).wait()

    fire_idx(0, 0)
    wait_idx(0)
    pltpu.async_copy(x_hbm.at[src0], rows0, sem_g0)
    fire_idx(1, 1)

    def half(j, p):
        q = 1 - p

        @pl.when(j + 1 < NCHUNK)
        def _():
            wait_idx(q)
            pltpu.async_copy(x_hbm.at[srcs[q]], rows[q], sem_g[q])

        pltpu.make_async_copy(x_hbm.at[srcs[p]], rows[p], sem_g[p]).wait()
        pltpu.sync_copy(rows[p], acc_sh.at[dsts[p]], add=True)

        @pl.when(j + 2 < NCHUNK)
        def _():
            fire_idx(j + 2, p)

    def step(t, c):
        half(2 * t, 0)
        half(2 * t + 1, 1)
        return c

    with jax.named_scope("sc_phase1_features"):
        lax.fori_loop(0, NCHUNK // 2, step, 0)
    plsc.subcore_barrier()

    pltpu.sync_copy(acc_sh.at[rslice], acc_out.at[cid, rslice])
    pltpu.sync_copy(zacc.at[rslice], acc_sh.at[rslice])
    pltpu.sync_copy(ones_hbm, rows0)
    plsc.subcore_barrier()

    fire_one(0, 0, dst_h, pdst_h, dsts)

    def dhalf(j, p):
        q = 1 - p

        @pl.when(j + 1 < NCHUNK)
        def _():
            fire_one(j + 1, q, dst_h, pdst_h, dsts)

        pltpu.make_async_copy(dst_h.at[pl.ds(0, K)], dsts[p], sem_i[p]).wait()
        pltpu.sync_copy(rows0, acc_sh.at[dsts[p]], add=True)

    def dstep(t, c):
        dhalf(2 * t, 0)
        dhalf(2 * t + 1, 1)
        return c

    with jax.named_scope("sc_phase2_degrees"):
        lax.fori_loop(0, NCHUNK // 2, dstep, 0)
    plsc.subcore_barrier()
    pltpu.sync_copy(acc_sh.at[rslice], deg_out.at[cid, rslice])


def _tc_body(acc_ref, deg_ref, batch_ref, wg_ref, bg_ref, w1_ref, b1_ref,
             w2_ref, b2_ref, out_ref):
    prec = lax.Precision.HIGHEST
    acc = acc_ref[0] + acc_ref[1]
    deg = deg_ref[0, :, 0:1] + deg_ref[1, :, 0:1]
    degc = jnp.maximum(deg, 1.0)
    agg = acc / degc
    h = jnp.maximum(
        jnp.dot(agg, wg_ref[...], precision=prec,
                preferred_element_type=jnp.float32) + bg_ref[...], 0.0)

    onehot = (batch_ref[...] ==
              lax.broadcasted_iota(jnp.int32, (G, N2), 0)).astype(jnp.float32)
    pooled = jnp.dot(onehot, h, precision=prec,
                     preferred_element_type=jnp.float32)
    cnt = jnp.dot(onehot, jnp.full((N2, 1), 1.0, jnp.float32),
                  precision=prec, preferred_element_type=jnp.float32)
    gmean = pooled / jnp.maximum(cnt, 1.0)

    h1 = jnp.maximum(
        jnp.dot(gmean, w1_ref[...], precision=prec,
                preferred_element_type=jnp.float32) + b1_ref[...], 0.0)
    out_ref[...] = jnp.dot(h1, w2_ref[...], precision=prec,
                           preferred_element_type=jnp.float32) + b2_ref[...]


@functools.lru_cache(maxsize=1)
def _sc_aggregate():
    return pl.kernel(
        _sc_body,
        out_type=(jax.ShapeDtypeStruct((NCORE, N2, D), jnp.float32),
                  jax.ShapeDtypeStruct((NCORE, N2, D), jnp.float32)),
        mesh=plsc.VectorSubcoreMesh(core_axis_name="c",
                                    subcore_axis_name="s"),
        scratch_types=[
            pltpu.VMEM_SHARED((N2, D), jnp.float32),
            pltpu.VMEM((K,), jnp.int32),
            pltpu.VMEM((K,), jnp.int32),
            pltpu.VMEM((K,), jnp.int32),
            pltpu.VMEM((K,), jnp.int32),
            pltpu.VMEM((K, D), jnp.float32),
            pltpu.VMEM((K, D), jnp.float32),
            pltpu.SemaphoreType.DMA,
            pltpu.SemaphoreType.DMA,
            pltpu.SemaphoreType.DMA,
            pltpu.SemaphoreType.DMA,
        ],
    )


_tc_head = pl.pallas_call(
    _tc_body,
    out_shape=jax.ShapeDtypeStruct((G, C), jnp.float32),
)


def kernel(x, edge_index, batch, W_gnn, b_gnn, W1, b1, W2, b2):
    rpt = (N2 - N) // NSUB
    pad_src = jnp.broadcast_to(
        (jnp.arange(PPT, dtype=jnp.int32) * 37) % N, (NT, PPT))
    tile_in_core = (jnp.arange(NT, dtype=jnp.int32) % NSUB)
    pad_dst = (N + tile_in_core[:, None] * rpt
               + jnp.arange(PPT, dtype=jnp.int32)[None, :] % rpt)
    zacc = jnp.zeros((N2, D), jnp.float32)
    ones = jnp.ones((K, D), jnp.float32)

    acc_part, deg_part = _sc_aggregate()(
        x, edge_index[0], edge_index[1], pad_src.reshape(-1),
        pad_dst.astype(jnp.int32).reshape(-1), zacc, ones)

    batch2 = jnp.concatenate(
        [batch, jnp.full((N2 - N,), G, jnp.int32)]).reshape(1, N2)
    return _tc_head(acc_part, deg_part, batch2,
                    W_gnn, b_gnn.reshape(1, D), W1, b1.reshape(1, H),
                    W2, b2.reshape(1, C))


# speedup vs baseline: 9.8092x; 1.0424x over previous
import functools

import jax
import jax.numpy as jnp
from jax import lax
from jax.experimental import pallas as pl
from jax.experimental.pallas import tpu as pltpu
from jax.experimental.pallas import tpu_sc as plsc

N = 10000
E = 320000
D = 128
H = 128
C = 10
G = 64

NCORE = 2
NSUB = 16
NT = NCORE * NSUB
K = 128
EPT = 10240
NCHUNK = EPT // K
IB = 8
NBLK = NCHUNK // IB
E_PAD = EPT * NT
N2 = 10240
DEGW = 16


RPT = E // NT
NFULL = RPT // K
RTAIL = RPT - NFULL * K
PPT = EPT - RPT
RSLC = N2 // NSUB


def _sc_body(x_hbm, ei_h, psrc_h, pdst_h, zacc, ones_hbm,
             acc_out, deg_out,
             acc_sh, src0, src1, dst0, dst1, rows0, rows1,
             sem_i0, sem_i1, sem_g0, sem_g1):
    cid = lax.axis_index("c")
    sid = lax.axis_index("s")
    wid = cid * NSUB + sid
    srcs = (src0, src1)
    dsts = (dst0, dst1)
    rows = (rows0, rows1)
    sem_i = (sem_i0, sem_i1)
    sem_g = (sem_g0, sem_g1)
    rslice = pl.ds(sid * RSLC, RSLC)

    pltpu.sync_copy(zacc.at[rslice], acc_sh.at[rslice])
    plsc.subcore_barrier()

    def fire_one(j, p, base, pad_h, bufs):
        @pl.when(j < NFULL)
        def _():
            pltpu.async_copy(ei_h.at[pl.ds(base + wid * RPT + j * K, K)],
                             bufs[p], sem_i[p])

        @pl.when(j == NFULL)
        def _():
            pltpu.async_copy(
                ei_h.at[pl.ds(base + wid * RPT + NFULL * K, RTAIL)],
                bufs[p].at[pl.ds(0, RTAIL)], sem_i[p])
            pltpu.async_copy(pad_h.at[pl.ds(wid * PPT, K - RTAIL)],
                             bufs[p].at[pl.ds(RTAIL, K - RTAIL)], sem_i[p])

        @pl.when(j == NFULL + 1)
        def _():
            pltpu.async_copy(pad_h.at[pl.ds(wid * PPT + K - RTAIL, K)],
                             bufs[p], sem_i[p])

    def fire_idx(j, p):
        fire_one(j, p, 0, psrc_h, srcs)
        fire_one(j, p, E, pdst_h, dsts)

    def wait_idx(p):
        pltpu.make_async_copy(ei_h.at[pl.ds(0, K)], srcs[p], sem_i[p]).wait()
        pltpu.make_async_copy(ei_h.at[pl.ds(0, K)], dsts[p], sem_i[p]).wait()

    fire_idx(0, 0)
    wait_idx(0)
    pltpu.async_copy(x_hbm.at[src0], rows0, sem_g0)
    fire_idx(1, 1)

    def half(j, p):
        q = 1 - p

        @pl.when(j + 1 < NCHUNK)
        def _():
            wait_idx(q)
            pltpu.async_copy(x_hbm.at[srcs[q]], rows[q], sem_g[q])

        pltpu.make_async_copy(x_hbm.at[srcs[p]], rows[p], sem_g[p]).wait()
        pltpu.sync_copy(rows[p], acc_sh.at[dsts[p]], add=True)

        @pl.when(j + 2 < NCHUNK)
        def _():
            fire_idx(j + 2, p)

    def step(t, c):
        half(2 * t, 0)
        half(2 * t + 1, 1)
        return c

    with jax.named_scope("sc_phase1_features"):
        lax.fori_loop(0, NCHUNK // 2, step, 0)
    plsc.subcore_barrier()

    pltpu.sync_copy(acc_sh.at[rslice], acc_out.at[cid, rslice])
    pltpu.sync_copy(zacc.at[rslice], acc_sh.at[rslice])
    pltpu.sync_copy(ones_hbm, rows0)
    plsc.subcore_barrier()

    fire_one(0, 0, E, pdst_h, dsts)

    def dhalf(j, p):
        q = 1 - p

        @pl.when(j + 1 < NCHUNK)
        def _():
            fire_one(j + 1, q, E, pdst_h, dsts)

        pltpu.make_async_copy(ei_h.at[pl.ds(0, K)], dsts[p], sem_i[p]).wait()
        pltpu.sync_copy(rows0, acc_sh.at[dsts[p]], add=True)

    def dstep(t, c):
        dhalf(2 * t, 0)
        dhalf(2 * t + 1, 1)
        return c

    with jax.named_scope("sc_phase2_degrees"):
        lax.fori_loop(0, NCHUNK // 2, dstep, 0)
    plsc.subcore_barrier()
    pltpu.sync_copy(acc_sh.at[rslice], deg_out.at[cid, rslice])


def _tc_body(acc_ref, deg_ref, batch_ref, wg_ref, bg_ref, w1_ref, b1_ref,
             w2_ref, b2_ref, out_ref):
    prec = lax.Precision.HIGHEST
    acc = acc_ref[0] + acc_ref[1]
    deg = deg_ref[0, :, 0:1] + deg_ref[1, :, 0:1]
    degc = jnp.maximum(deg, 1.0)
    agg = acc / degc
    h = jnp.maximum(
        jnp.dot(agg, wg_ref[...], precision=prec,
                preferred_element_type=jnp.float32) + bg_ref[...], 0.0)

    onehot = (batch_ref[...] ==
              lax.broadcasted_iota(jnp.int32, (G, N2), 0)).astype(jnp.float32)
    pooled = jnp.dot(onehot, h, precision=prec,
                     preferred_element_type=jnp.float32)
    cnt = jnp.dot(onehot, jnp.full((N2, 1), 1.0, jnp.float32),
                  precision=prec, preferred_element_type=jnp.float32)
    gmean = pooled / jnp.maximum(cnt, 1.0)

    h1 = jnp.maximum(
        jnp.dot(gmean, w1_ref[...], precision=prec,
                preferred_element_type=jnp.float32) + b1_ref[...], 0.0)
    out_ref[...] = jnp.dot(h1, w2_ref[...], precision=prec,
                           preferred_element_type=jnp.float32) + b2_ref[...]


@functools.lru_cache(maxsize=1)
def _sc_aggregate():
    return pl.kernel(
        _sc_body,
        out_type=(jax.ShapeDtypeStruct((NCORE, N2, D), jnp.float32),
                  jax.ShapeDtypeStruct((NCORE, N2, D), jnp.float32)),
        mesh=plsc.VectorSubcoreMesh(core_axis_name="c",
                                    subcore_axis_name="s"),
        scratch_types=[
            pltpu.VMEM_SHARED((N2, D), jnp.float32),
            pltpu.VMEM((K,), jnp.int32),
            pltpu.VMEM((K,), jnp.int32),
            pltpu.VMEM((K,), jnp.int32),
            pltpu.VMEM((K,), jnp.int32),
            pltpu.VMEM((K, D), jnp.float32),
            pltpu.VMEM((K, D), jnp.float32),
            pltpu.SemaphoreType.DMA,
            pltpu.SemaphoreType.DMA,
            pltpu.SemaphoreType.DMA,
            pltpu.SemaphoreType.DMA,
        ],
    )


_tc_head = pl.pallas_call(
    _tc_body,
    out_shape=jax.ShapeDtypeStruct((G, C), jnp.float32),
)


def kernel(x, edge_index, batch, W_gnn, b_gnn, W1, b1, W2, b2):
    rpt = (N2 - N) // NSUB
    pad_src = jnp.broadcast_to(
        (jnp.arange(PPT, dtype=jnp.int32) * 37) % N, (NT, PPT))
    tile_in_core = (jnp.arange(NT, dtype=jnp.int32) % NSUB)
    pad_dst = (N + tile_in_core[:, None] * rpt
               + jnp.arange(PPT, dtype=jnp.int32)[None, :] % rpt)
    zacc = jnp.zeros((N2, D), jnp.float32)
    ones = jnp.ones((K, D), jnp.float32)

    acc_part, deg_part = _sc_aggregate()(
        x, edge_index.reshape(-1), pad_src.reshape(-1),
        pad_dst.astype(jnp.int32).reshape(-1), zacc, ones)

    batch2 = jnp.concatenate(
        [batch, jnp.full((N2 - N,), G, jnp.int32)]).reshape(1, N2)
    return _tc_head(acc_part, deg_part, batch2,
                    W_gnn, b_gnn.reshape(1, D), W1, b1.reshape(1, H),
                    W2, b2.reshape(1, C))


# speedup vs baseline: 9.9115x; 1.0104x over previous
import functools

import jax
import jax.numpy as jnp
from jax import lax
from jax.experimental import pallas as pl
from jax.experimental.pallas import tpu as pltpu
from jax.experimental.pallas import tpu_sc as plsc

N = 10000
E = 320000
D = 128
H = 128
C = 10
G = 64

NCORE = 2
NSUB = 16
NT = NCORE * NSUB
K = 128
EPT = 10240
NCHUNK = EPT // K
N2 = 10240

RPT = E // NT
NFULL = RPT // K
RTAIL = RPT - NFULL * K
PPT = EPT - RPT
RSLC = N2 // NSUB


def _fire_one(j, p, wid, base, ei_h, pad_h, bufs, sem_i):
    @pl.when(j < NFULL)
    def _():
        pltpu.async_copy(ei_h.at[pl.ds(base + wid * RPT + j * K, K)],
                         bufs[p], sem_i[p])

    @pl.when(j == NFULL)
    def _():
        pltpu.async_copy(
            ei_h.at[pl.ds(base + wid * RPT + NFULL * K, RTAIL)],
            bufs[p].at[pl.ds(0, RTAIL)], sem_i[p])
        pltpu.async_copy(pad_h.at[pl.ds(wid * PPT, K - RTAIL)],
                         bufs[p].at[pl.ds(RTAIL, K - RTAIL)], sem_i[p])

    @pl.when(j == NFULL + 1)
    def _():
        pltpu.async_copy(pad_h.at[pl.ds(wid * PPT + K - RTAIL, K)],
                         bufs[p], sem_i[p])


def _sc_p1_body(x_hbm, ei_h, psrc_h, pdst_h, zacc, acc_out,
                acc_sh, src0, src1, dst0, dst1, rows0, rows1,
                sem_i0, sem_i1, sem_g0, sem_g1):
    cid = lax.axis_index("c")
    sid = lax.axis_index("s")
    wid = cid * NSUB + sid
    srcs = (src0, src1)
    dsts = (dst0, dst1)
    rows = (rows0, rows1)
    sem_i = (sem_i0, sem_i1)
    sem_g = (sem_g0, sem_g1)
    rslice = pl.ds(sid * RSLC, RSLC)

    pltpu.sync_copy(zacc.at[rslice], acc_sh.at[rslice])
    plsc.subcore_barrier()

    def fire_idx(j, p):
        _fire_one(j, p, wid, 0, ei_h, psrc_h, srcs, sem_i)
        _fire_one(j, p, wid, E, ei_h, pdst_h, dsts, sem_i)

    def wait_idx(p):
        pltpu.make_async_copy(ei_h.at[pl.ds(0, K)], srcs[p], sem_i[p]).wait()
        pltpu.make_async_copy(ei_h.at[pl.ds(0, K)], dsts[p], sem_i[p]).wait()

    fire_idx(0, 0)
    wait_idx(0)
    pltpu.async_copy(x_hbm.at[src0], rows0, sem_g0)
    fire_idx(1, 1)

    def half(j, p):
        q = 1 - p

        @pl.when(j + 1 < NCHUNK)
        def _():
            wait_idx(q)
            pltpu.async_copy(x_hbm.at[srcs[q]], rows[q], sem_g[q])

        pltpu.make_async_copy(x_hbm.at[srcs[p]], rows[p], sem_g[p]).wait()
        pltpu.sync_copy(rows[p], acc_sh.at[dsts[p]], add=True)

        @pl.when(j + 2 < NCHUNK)
        def _():
            fire_idx(j + 2, p)

    def step(t, c):
        half(2 * t, 0)
        half(2 * t + 1, 1)
        return c

    with jax.named_scope("sc_phase1_features"):
        lax.fori_loop(0, NCHUNK // 2, step, 0)
    plsc.subcore_barrier()
    pltpu.sync_copy(acc_sh.at[rslice], acc_out.at[cid, rslice])


def _sc_p2_body(ei_h, pdst_h, zacc, ones_hbm, deg_out,
                acc_sh, dst0, dst1, ones_v, sem_i0, sem_i1):
    cid = lax.axis_index("c")
    sid = lax.axis_index("s")
    wid = cid * NSUB + sid
    dsts = (dst0, dst1)
    sem_i = (sem_i0, sem_i1)
    rslice = pl.ds(sid * RSLC, RSLC)

    pltpu.sync_copy(zacc.at[rslice], acc_sh.at[rslice])
    pltpu.sync_copy(ones_hbm, ones_v)
    plsc.subcore_barrier()

    _fire_one(0, 0, wid, E, ei_h, pdst_h, dsts, sem_i)

    def dhalf(j, p):
        q = 1 - p

        @pl.when(j + 1 < NCHUNK)
        def _():
            _fire_one(j + 1, q, wid, E, ei_h, pdst_h, dsts, sem_i)

        pltpu.make_async_copy(ei_h.at[pl.ds(0, K)], dsts[p], sem_i[p]).wait()
        pltpu.sync_copy(ones_v, acc_sh.at[dsts[p]], add=True)

    def dstep(t, c):
        dhalf(2 * t, 0)
        dhalf(2 * t + 1, 1)
        return c

    with jax.named_scope("sc_phase2_degrees"):
        lax.fori_loop(0, NCHUNK // 2, dstep, 0)
    plsc.subcore_barrier()
    pltpu.sync_copy(acc_sh.at[rslice], deg_out.at[cid, rslice])


def _tc_a_body(acc_ref, wg_ref, y_ref):
    y_ref[...] = jnp.dot(acc_ref[0] + acc_ref[1], wg_ref[...],
                         precision=lax.Precision.HIGHEST,
                         preferred_element_type=jnp.float32)


def _tc_b_body(y_ref, deg_ref, batch_ref, bg_ref, w1_ref, b1_ref,
               w2_ref, b2_ref, out_ref):
    prec = lax.Precision.HIGHEST
    deg = deg_ref[0, :, 0:1] + deg_ref[1, :, 0:1]
    degc = jnp.maximum(deg, 1.0)
    h = jnp.maximum(y_ref[...] / degc + bg_ref[...], 0.0)

    onehot = (batch_ref[...] ==
              lax.broadcasted_iota(jnp.int32, (G, N2), 0)).astype(jnp.float32)
    pooled = jnp.dot(onehot, h, precision=prec,
                     preferred_element_type=jnp.float32)
    cnt = jnp.dot(onehot, jnp.full((N2, 1), 1.0, jnp.float32),
                  precision=prec, preferred_element_type=jnp.float32)
    gmean = pooled / jnp.maximum(cnt, 1.0)

    h1 = jnp.maximum(
        jnp.dot(gmean, w1_ref[...], precision=prec,
                preferred_element_type=jnp.float32) + b1_ref[...], 0.0)
    out_ref[...] = jnp.dot(h1, w2_ref[...], precision=prec,
                           preferred_element_type=jnp.float32) + b2_ref[...]


@functools.lru_cache(maxsize=1)
def _sc_p1():
    return pl.kernel(
        _sc_p1_body,
        out_type=jax.ShapeDtypeStruct((NCORE, N2, D), jnp.float32),
        mesh=plsc.VectorSubcoreMesh(core_axis_name="c",
                                    subcore_axis_name="s"),
        scratch_types=[
            pltpu.VMEM_SHARED((N2, D), jnp.float32),
            pltpu.VMEM((K,), jnp.int32),
            pltpu.VMEM((K,), jnp.int32),
            pltpu.VMEM((K,), jnp.int32),
            pltpu.VMEM((K,), jnp.int32),
            pltpu.VMEM((K, D), jnp.float32),
            pltpu.VMEM((K, D), jnp.float32),
            pltpu.SemaphoreType.DMA,
            pltpu.SemaphoreType.DMA,
            pltpu.SemaphoreType.DMA,
            pltpu.SemaphoreType.DMA,
        ],
    )


@functools.lru_cache(maxsize=1)
def _sc_p2():
    return pl.kernel(
        _sc_p2_body,
        out_type=jax.ShapeDtypeStruct((NCORE, N2, D), jnp.float32),
        mesh=plsc.VectorSubcoreMesh(core_axis_name="c",
                                    subcore_axis_name="s"),
        scratch_types=[
            pltpu.VMEM_SHARED((N2, D), jnp.float32),
            pltpu.VMEM((K,), jnp.int32),
            pltpu.VMEM((K,), jnp.int32),
            pltpu.VMEM((K, D), jnp.float32),
            pltpu.SemaphoreType.DMA,
            pltpu.SemaphoreType.DMA,
        ],
    )


_tc_a = pl.pallas_call(
    _tc_a_body,
    out_shape=jax.ShapeDtypeStruct((N2, D), jnp.float32),
)

_tc_b = pl.pallas_call(
    _tc_b_body,
    out_shape=jax.ShapeDtypeStruct((G, C), jnp.float32),
)


def kernel(x, edge_index, batch, W_gnn, b_gnn, W1, b1, W2, b2):
    rpt = (N2 - N) // NSUB
    pad_src = jnp.broadcast_to(
        (jnp.arange(PPT, dtype=jnp.int32) * 37) % N, (NT, PPT)).reshape(-1)
    tile_in_core = (jnp.arange(NT, dtype=jnp.int32) % NSUB)
    pad_dst = (N + tile_in_core[:, None] * rpt
               + jnp.arange(PPT, dtype=jnp.int32)[None, :] % rpt
               ).astype(jnp.int32).reshape(-1)
    zacc = jnp.zeros((N2, D), jnp.float32)
    ones = jnp.ones((K, D), jnp.float32)
    ei = edge_index.reshape(-1)

    acc_part = _sc_p1()(x, ei, pad_src, pad_dst, zacc)
    deg_part = _sc_p2()(ei, pad_dst, zacc, ones)
    y = _tc_a(acc_part, W_gnn)

    batch2 = jnp.concatenate(
        [batch, jnp.full((N2 - N,), G, jnp.int32)]).reshape(1, N2)
    return _tc_b(y, deg_part, batch2, b_gnn.reshape(1, D),
                 W1, b1.reshape(1, H), W2, b2.reshape(1, C))
